# baseline (speedup 1.0000x reference)
import functools

import jax
import jax.numpy as jnp
from jax import lax
from jax.experimental import pallas as pl
from jax.experimental.pallas import tpu as pltpu
from jax.experimental.pallas import tpu_sc as plsc

D = 128
N = 10000
E = 160000

NC = 2
NS = 16
NW = NC * NS
CH = 128
NCHUNK = E // CH
CHUNKS_PER_TILE = (NCHUNK + NW - 1) // NW
NPER = 640
NPER_LAST = N - (NS - 1) * NPER
CP = 80
GMAX = CHUNKS_PER_TILE

_EPS = 1e-5


def _tab_body(pc_ref, w1_ref, b1_ref, q_ref, k_ref):
    xh = lax.dot_general(pc_ref[...], w1_ref[...], (((1,), (1,)), ((), ())),
                         preferred_element_type=jnp.float32)
    xh = xh + b1_ref[...]
    q_ref[...] = xh[:, :D]
    k_ref[...] = xh[:, D:]


def _make_tables(poly_conn, W1, b1):
    BN_ = 1000
    grid = (N // BN_,)
    return pl.pallas_call(
        _tab_body,
        grid=grid,
        in_specs=[
            pl.BlockSpec((BN_, D), lambda i: (i, 0)),
            pl.BlockSpec((2 * D, D), lambda i: (0, 0)),
            pl.BlockSpec((1, 2 * D), lambda i: (0, 0)),
        ],
        out_specs=[
            pl.BlockSpec((BN_, D), lambda i: (i, 0)),
            pl.BlockSpec((BN_, D), lambda i: (i, 0)),
        ],
        out_shape=[
            jax.ShapeDtypeStruct((N, D), jnp.float32),
            jax.ShapeDtypeStruct((N, D), jnp.float32),
        ],
    )(poly_conn, W1, b1)


@functools.cache
def _sc_mesh():
    return plsc.VectorSubcoreMesh(core_axis_name="c", subcore_axis_name="s",
                                  num_cores=NC, num_subcores=NS)


def _tile_range(wid, nchunk):
    base, rem = nchunk // NW, nchunk % NW
    start = wid * base + jnp.minimum(wid, rem)
    cnt = base + jnp.where(wid < rem, 1, 0)
    return start, cnt


def _gmax(nchunk):
    return nchunk // NW + (1 if nchunk % NW else 0)


def _vcopy16(dst_ref, dst_off, src_ref, src_off, n):
    for k in range(n // 16):
        dst_ref[pl.ds(dst_off + k * 16, 16)] = src_ref[pl.ds(src_off + k * 16, 16)]


@functools.cache
def _sc_gather_kernel(nchunk):
    gmax = _gmax(nchunk)
    return functools.partial(
        pl.kernel,
        out_type=jax.ShapeDtypeStruct((nchunk * CH, D), jnp.float32),
        mesh=_sc_mesh(),
        scratch_types=[
            pltpu.VMEM((gmax * CH,), jnp.int32),
            pltpu.VMEM((gmax * CH,), jnp.int32),
            pltpu.VMEM((CH,), jnp.int32),
            pltpu.VMEM((CH,), jnp.int32),
            pltpu.VMEM((CH,), jnp.int32),
            pltpu.VMEM((CH,), jnp.int32),
            pltpu.VMEM((CH, D), jnp.float32),
            pltpu.VMEM((CH, D), jnp.float32),
            pltpu.VMEM((CH, D), jnp.float32),
            pltpu.VMEM((CH, D), jnp.float32),
            pltpu.SemaphoreType.DMA,
            pltpu.SemaphoreType.DMA,
            pltpu.SemaphoreType.DMA,
            pltpu.SemaphoreType.DMA,
        ],
    )(_make_gather_body(nchunk))


def _sc_gather(qtab, ktab, dst_p, src_p, nchunk):
    return _sc_gather_kernel(nchunk)(qtab, ktab, dst_p, src_p)


def _make_gather_body(nchunk):
    gmax = _gmax(nchunk)

    def body_fn(qtab_hbm, ktab_hbm, dst_hbm, src_hbm, out_hbm,
                idall_d, idall_s, idx_d0, idx_s0, idx_d1, idx_s1,
                bq0, bk0, bq1, bk1,
                sem_g0, sem_g1, sem_w0, sem_w1):
        wid = lax.axis_index("s") * NC + lax.axis_index("c")
        start, cnt = _tile_range(wid, nchunk)
        ebase = start * CH

        pltpu.sync_copy(dst_hbm.at[pl.ds(ebase, gmax * CH)], idall_d)
        pltpu.sync_copy(src_hbm.at[pl.ds(ebase, gmax * CH)], idall_s)

        def issue(j, idx_d, idx_s, bq, bk, sem_g):
            @pl.when(j < cnt)
            def _():
                _vcopy16(idx_d, 0, idall_d, j * CH, CH)
                _vcopy16(idx_s, 0, idall_s, j * CH, CH)
                pltpu.async_copy(qtab_hbm.at[idx_d], bq, sem_g)
                pltpu.async_copy(ktab_hbm.at[idx_s], bk, sem_g)

        def process(j, idx_d, bq, bk, sem_g, sem_w):
            @pl.when((j >= 0) & (j < cnt))
            def _():
                @pl.when(j >= 2)
                def _():
                    pltpu.make_async_copy(
                        bq, out_hbm.at[pl.ds((start + j - 2) * CH, CH)], sem_w
                    ).wait()

                pltpu.make_async_copy(qtab_hbm.at[idx_d], bq, sem_g).wait()
                pltpu.make_async_copy(qtab_hbm.at[idx_d], bk, sem_g).wait()

                def add_body(i, carry2):
                    for k in range(D // 16):
                        sl = pl.ds(k * 16, 16)
                        bq[i, sl] = bq[i, sl] + bk[i, sl]
                    return carry2

                lax.fori_loop(0, CH, add_body, 0)
                pltpu.async_copy(
                    bq, out_hbm.at[pl.ds((start + j) * CH, CH)], sem_w)

        def body(t, carry):
            j0 = 2 * t
            j1 = 2 * t + 1
            issue(j0, idx_d0, idx_s0, bq0, bk0, sem_g0)
            process(j0 - 1, idx_d1, bq1, bk1, sem_g1, sem_w1)
            issue(j1, idx_d1, idx_s1, bq1, bk1, sem_g1)
            process(j0, idx_d0, bq0, bk0, sem_g0, sem_w0)
            return carry

        lax.fori_loop(0, gmax // 2 + 1, body, 0)
        pltpu.make_async_copy(
            bq0, out_hbm.at[pl.ds(ebase, CH)], sem_w0).wait()
        pltpu.make_async_copy(
            bq1, out_hbm.at[pl.ds(ebase, CH)], sem_w1).wait()

    return body_fn


_BE = 2000
_UNIT = 16000
_CHUNK_UNITS = (3, 3, 2, 1, 1)
KCH = len(_CHUNK_UNITS)


def _edge_body(pc_ref, g_ref, w1_ref, b1_ref, w2_ref, b2_ref,
               pre_ref, stats_ref):
    i = pl.program_id(0)
    pc = pc_ref[...]
    xh = lax.dot_general(pc, w1_ref[...], (((1,), (1,)), ((), ())),
                         preferred_element_type=jnp.float32)
    xh = xh + b1_ref[...]
    qh = xh[:, :D]
    kh = xh[:, D:]
    conn = jnp.maximum(g_ref[...] * qh + kh, 0.0)
    pre = lax.dot_general(conn, w2_ref[...], (((1,), (1,)), ((), ())),
                          preferred_element_type=jnp.float32)
    pre = pre + b2_ref[...] + pc
    pre_ref[...] = pre.astype(jnp.bfloat16)

    @pl.when(i == 0)
    def _():
        stats_ref[...] = jnp.zeros((8, D), jnp.float32)

    s1 = jnp.sum(pre, axis=0, keepdims=True)
    s2 = jnp.sum(pre * pre, axis=0, keepdims=True)
    stats_ref[0:1, :] = stats_ref[0:1, :] + s1
    stats_ref[1:2, :] = stats_ref[1:2, :] + s2


def _edge_stage_chunk(poly_conn, Gk, W1, b1, W2, b2, pre_full, b0, nb):
    in_specs = [
        pl.BlockSpec((_BE, D), lambda i: (i + b0, 0)),
        pl.BlockSpec((_BE, D), lambda i: (i, 0)),
        pl.BlockSpec((2 * D, D), lambda i: (0, 0)),
        pl.BlockSpec((1, 2 * D), lambda i: (0, 0)),
        pl.BlockSpec((D, D), lambda i: (0, 0)),
        pl.BlockSpec((1, D), lambda i: (0, 0)),
    ]
    inputs = (poly_conn, Gk, W1, b1, W2, b2)
    body = _edge_body
    kwargs = {}
    if b0 > 0:
        in_specs.append(pl.BlockSpec((_BE, D), lambda i: (i + b0, 0)))
        inputs = inputs + (pre_full,)
        kwargs["input_output_aliases"] = {6: 0}

        def body(pc, g, w1, b1_, w2, b2_, _pre_in, pre, st):
            _edge_body(pc, g, w1, b1_, w2, b2_, pre, st)

    return pl.pallas_call(
        body,
        grid=(nb,),
        in_specs=in_specs,
        out_specs=[
            pl.BlockSpec((_BE, D), lambda i: (i + b0, 0)),
            pl.BlockSpec((8, D), lambda i: (0, 0)),
        ],
        out_shape=[
            jax.ShapeDtypeStruct((E, D), jnp.bfloat16),
            jax.ShapeDtypeStruct((8, D), jnp.float32),
        ],
        **kwargs,
    )(*inputs)


def _bnrelu_body(pre_ref, st0, st1, st2, st3, st4, g_ref, b_ref, out_ref):
    stats = st0[...] + st1[...] + st2[...] + st3[...] + st4[...]
    inv_e = jnp.float32(1.0 / E)
    mean = stats[0:1, :] * inv_e
    ex2 = stats[1:2, :] * inv_e
    var = ex2 - mean * mean
    inv = lax.rsqrt(var + _EPS)
    scale = g_ref[...] * inv
    bias = b_ref[...] - mean * scale
    pre = pre_ref[...].astype(jnp.float32)
    out_ref[...] = jnp.maximum(pre * scale + bias, 0.0)


def _bnrelu_stage(pre, stats_list, cn_gamma, cn_beta):
    grid = (E // _BE,)
    return pl.pallas_call(
        _bnrelu_body,
        grid=grid,
        in_specs=[
            pl.BlockSpec((_BE, D), lambda i: (i, 0)),
        ] + [pl.BlockSpec((8, D), lambda i: (0, 0)) for _ in range(KCH)] + [
            pl.BlockSpec((1, D), lambda i: (0, 0)),
            pl.BlockSpec((1, D), lambda i: (0, 0)),
        ],
        out_specs=pl.BlockSpec((_BE, D), lambda i: (i, 0)),
        out_shape=jax.ShapeDtypeStruct((E, D), jnp.float32),
    )(pre, *stats_list, cn_gamma, cn_beta)


@functools.cache
def _sc_scatter_kernel():
    return functools.partial(
        pl.kernel,
        out_type=jax.ShapeDtypeStruct((NC, N, D), jnp.float32),
        mesh=_sc_mesh(),
        scratch_types=[
            pltpu.VMEM((GMAX * CH,), jnp.int32),
            pltpu.VMEM((CH,), jnp.int32),
            pltpu.VMEM((CH,), jnp.int32),
            pltpu.VMEM((CH, D), jnp.float32),
            pltpu.VMEM((CH, D), jnp.float32),
            pltpu.VMEM((CP, D), jnp.float32),
            pltpu.VMEM_SHARED((N, D), jnp.float32),
            pltpu.SemaphoreType.DMA,
            pltpu.SemaphoreType.DMA,
            pltpu.SemaphoreType.DMA,
            pltpu.SemaphoreType.DMA,
        ],
    )(_sc_scatter_body)


def _sc_scatter(conn2, dst_p):
    return _sc_scatter_kernel()(conn2, dst_p)


def _sc_scatter_body(conn2_hbm, dst_hbm, out_hbm,
                     idall, idx_v0, idx_v1, rows0, rows1, stage_v, acc_sh,
                     sem_l0, sem_l1, sem_a0, sem_a1):
    cid = lax.axis_index("c")
    sid = lax.axis_index("s")
    wid = sid * NC + cid
    nbase = pl.multiple_of(sid * NPER, 8)
    start, cnt = _tile_range(wid, NCHUNK)
    ebase = start * CH

    zero16 = jnp.zeros((16,), jnp.float32)

    def zero_body(i, carry):
        for k in range(D // 16):
            stage_v[i, pl.ds(k * 16, 16)] = zero16
        return carry

    lax.fori_loop(0, CP, zero_body, 0)

    @pl.when(sid < NS - 1)
    def _():
        for t in range(NPER // CP):
            pltpu.sync_copy(stage_v, acc_sh.at[pl.ds(nbase + t * CP, CP)])

    @pl.when(sid == NS - 1)
    def _():
        for t in range(NPER_LAST // CP):
            pltpu.sync_copy(stage_v, acc_sh.at[pl.ds(nbase + t * CP, CP)])

    pltpu.sync_copy(dst_hbm.at[pl.ds(ebase, GMAX * CH)], idall)
    plsc.subcore_barrier()

    def issue(j, rows, sem_l, idx_v, sem_a):
        @pl.when(j < cnt)
        def _():
            @pl.when(j >= 2)
            def _():
                pltpu.make_async_copy(rows, acc_sh.at[idx_v], sem_a).wait()

            pltpu.async_copy(
                conn2_hbm.at[pl.ds((start + j) * CH, CH)], rows, sem_l)

    def process(j, rows, sem_l, idx_v, sem_a):
        @pl.when((j >= 0) & (j < cnt))
        def _():
            pltpu.make_async_copy(
                conn2_hbm.at[pl.ds(ebase, CH)], rows, sem_l).wait()
            _vcopy16(idx_v, 0, idall, j * CH, CH)
            pltpu.async_copy(rows, acc_sh.at[idx_v], sem_a, add=True)

    def body(t, carry):
        j0 = 2 * t
        j1 = 2 * t + 1
        issue(j0, rows0, sem_l0, idx_v0, sem_a0)
        process(j0 - 1, rows1, sem_l1, idx_v1, sem_a1)
        issue(j1, rows1, sem_l1, idx_v1, sem_a1)
        process(j0, rows0, sem_l0, idx_v0, sem_a0)
        return carry

    lax.fori_loop(0, GMAX // 2 + 1, body, 0)
    pltpu.make_async_copy(rows0, acc_sh.at[idx_v0], sem_a0).wait()
    pltpu.make_async_copy(rows1, acc_sh.at[idx_v1], sem_a1).wait()
    plsc.subcore_barrier()

    @pl.when(sid < NS - 1)
    def _():
        for t in range(NPER // CP):
            off = nbase + t * CP
            pltpu.sync_copy(acc_sh.at[pl.ds(off, CP)], stage_v)
            pltpu.sync_copy(stage_v, out_hbm.at[cid, pl.ds(off, CP)])

    @pl.when(sid == NS - 1)
    def _():
        for t in range(NPER_LAST // CP):
            off = nbase + t * CP
            pltpu.sync_copy(acc_sh.at[pl.ds(off, CP)], stage_v)
            pltpu.sync_copy(stage_v, out_hbm.at[cid, pl.ds(off, CP)])


def _node_body(part_ref, x_ref, sdeg_ref, dc0_ref, dc1_ref,
               wf1_ref, bf1_ref, wf2_ref, bf2_ref,
               g1_ref, b1n_ref, g2_ref, b2n_ref, out_ref):
    nh = part_ref[0] + part_ref[1]
    sdeg = sdeg_ref[...]
    nh = nh * (dc0_ref[...] + sdeg * dc1_ref[...])
    h_res = nh + x_ref[...]

    m1 = jnp.mean(h_res, axis=0, keepdims=True)
    hc = h_res - m1
    v1 = jnp.mean(hc * hc, axis=0, keepdims=True)
    nh1 = g1_ref[...] * hc * lax.rsqrt(v1 + _EPS) + b1n_ref[...]

    t = lax.dot_general(nh1, wf1_ref[...], (((1,), (1,)), ((), ())),
                        preferred_element_type=jnp.float32)
    t = jnp.maximum(t + bf1_ref[...], 0.0)
    t2 = lax.dot_general(t, wf2_ref[...], (((1,), (1,)), ((), ())),
                         preferred_element_type=jnp.float32)
    h2 = t2 + bf2_ref[...] + h_res

    m2 = jnp.mean(h2, axis=0, keepdims=True)
    hc2 = h2 - m2
    v2 = jnp.mean(hc2 * hc2, axis=0, keepdims=True)
    out_ref[...] = g2_ref[...] * hc2 * lax.rsqrt(v2 + _EPS) + b2n_ref[...]


def _node_stage(parts, x, sdeg, dc0, dc1, Wf1, bf1, Wf2, bf2,
                n1_gamma, n1_beta, n2_gamma, n2_beta):
    return pl.pallas_call(
        _node_body,
        in_specs=[
            pl.BlockSpec((NC, N, D), lambda: (0, 0, 0)),
            pl.BlockSpec((N, D), lambda: (0, 0)),
            pl.BlockSpec((N, 1), lambda: (0, 0)),
            pl.BlockSpec((1, D), lambda: (0, 0)),
            pl.BlockSpec((1, D), lambda: (0, 0)),
            pl.BlockSpec((2 * D, D), lambda: (0, 0)),
            pl.BlockSpec((1, 2 * D), lambda: (0, 0)),
            pl.BlockSpec((D, 2 * D), lambda: (0, 0)),
            pl.BlockSpec((1, D), lambda: (0, 0)),
            pl.BlockSpec((1, D), lambda: (0, 0)),
            pl.BlockSpec((1, D), lambda: (0, 0)),
            pl.BlockSpec((1, D), lambda: (0, 0)),
            pl.BlockSpec((1, D), lambda: (0, 0)),
        ],
        out_specs=pl.BlockSpec((N, D), lambda: (0, 0)),
        out_shape=jax.ShapeDtypeStruct((N, D), jnp.float32),
    )(parts, x, sdeg, dc0, dc1, Wf1, bf1, Wf2, bf2,
      n1_gamma, n1_beta, n2_gamma, n2_beta)


def kernel(x, poly_conn, sqrt_deg, W1, b1, W2, b2, cn_gamma, cn_beta,
           deg_coef, Wf1, bf1, Wf2, bf2, n1_gamma, n1_beta, n2_gamma,
           n2_beta, poly_index):
    dst = poly_index[0].astype(jnp.int32)
    src = poly_index[1].astype(jnp.int32)
    pad = jnp.zeros((CH,), jnp.int32)
    dst_p = jnp.concatenate([dst, pad])
    src_p = jnp.concatenate([src, pad])

    b1r = b1.reshape(1, 2 * D)
    b2r = b2.reshape(1, D)
    bf1r = bf1.reshape(1, 2 * D)
    bf2r = bf2.reshape(1, D)
    cng = cn_gamma.reshape(1, D)
    cnb = cn_beta.reshape(1, D)
    dc0 = deg_coef[:, :, 0]
    dc1 = deg_coef[:, :, 1]

    qtab, ktab = _make_tables(poly_conn, W1, b1r)
    Gs = []
    off = 0
    for u in _CHUNK_UNITS:
        n_e = u * _UNIT
        Gs.append(_sc_gather(qtab, ktab, dst_p[off:off + n_e + CH],
                             src_p[off:off + n_e + CH], n_e // CH))
        off += n_e
    pre = None
    stats_list = []
    off = 0
    for k, u in enumerate(_CHUNK_UNITS):
        n_e = u * _UNIT
        pre, st = _edge_stage_chunk(poly_conn, Gs[k], W1, b1r, W2, b2r, pre,
                                    off // _BE, n_e // _BE)
        stats_list.append(st)
        off += n_e
    conn2 = _bnrelu_stage(pre, stats_list, cng, cnb)
    parts = _sc_scatter(conn2, dst_p)
    nh = _node_stage(parts, x, sqrt_deg, dc0, dc1, Wf1, bf1r, Wf2, bf2r,
                     n1_gamma.reshape(1, D), n1_beta.reshape(1, D),
                     n2_gamma.reshape(1, D), n2_beta.reshape(1, D))
    return nh, conn2


# speedup vs baseline: 1.0866x; 1.0866x over previous
import functools

import jax
import jax.numpy as jnp
from jax import lax
from jax.experimental import pallas as pl
from jax.experimental.pallas import tpu as pltpu
from jax.experimental.pallas import tpu_sc as plsc

D = 128
N = 10000
E = 160000

NC = 2
NS = 16
NW = NC * NS
CH = 128
NCHUNK = E // CH
CHUNKS_PER_TILE = (NCHUNK + NW - 1) // NW
NPER = 640
NPER_LAST = N - (NS - 1) * NPER
CP = 80
GMAX = CHUNKS_PER_TILE

_EPS = 1e-5


def _tab_body(pc_ref, w1_ref, b1_ref, q_ref, k_ref):
    xh = lax.dot_general(pc_ref[...], w1_ref[...], (((1,), (1,)), ((), ())),
                         preferred_element_type=jnp.float32)
    xh = xh + b1_ref[...]
    q_ref[...] = xh[:, :D]
    k_ref[...] = xh[:, D:]


def _make_tables(poly_conn, W1, b1):
    BN_ = 1000
    grid = (N // BN_,)
    return pl.pallas_call(
        _tab_body,
        grid=grid,
        in_specs=[
            pl.BlockSpec((BN_, D), lambda i: (i, 0)),
            pl.BlockSpec((2 * D, D), lambda i: (0, 0)),
            pl.BlockSpec((1, 2 * D), lambda i: (0, 0)),
        ],
        out_specs=[
            pl.BlockSpec((BN_, D), lambda i: (i, 0)),
            pl.BlockSpec((BN_, D), lambda i: (i, 0)),
        ],
        out_shape=[
            jax.ShapeDtypeStruct((N, D), jnp.float32),
            jax.ShapeDtypeStruct((N, D), jnp.float32),
        ],
    )(poly_conn, W1, b1)


@functools.cache
def _sc_mesh():
    return plsc.VectorSubcoreMesh(core_axis_name="c", subcore_axis_name="s",
                                  num_cores=NC, num_subcores=NS)


def _tile_range(wid, nchunk):
    base, rem = nchunk // NW, nchunk % NW
    start = wid * base + jnp.minimum(wid, rem)
    cnt = base + jnp.where(wid < rem, 1, 0)
    return start, cnt


def _gmax(nchunk):
    return nchunk // NW + (1 if nchunk % NW else 0)


def _vcopy16(dst_ref, dst_off, src_ref, src_off, n):
    for k in range(n // 16):
        dst_ref[pl.ds(dst_off + k * 16, 16)] = src_ref[pl.ds(src_off + k * 16, 16)]


@functools.cache
def _sc_gather_kernel(nchunk):
    gmax = _gmax(nchunk)
    return functools.partial(
        pl.kernel,
        out_type=jax.ShapeDtypeStruct((nchunk * CH, D), jnp.float32),
        mesh=_sc_mesh(),
        scratch_types=[
            pltpu.VMEM((gmax * CH,), jnp.int32),
            pltpu.VMEM((gmax * CH,), jnp.int32),
            pltpu.VMEM((CH,), jnp.int32),
            pltpu.VMEM((CH,), jnp.int32),
            pltpu.VMEM((CH,), jnp.int32),
            pltpu.VMEM((CH,), jnp.int32),
            pltpu.VMEM((CH, D), jnp.float32),
            pltpu.VMEM((CH, D), jnp.float32),
            pltpu.VMEM((CH, D), jnp.float32),
            pltpu.VMEM((CH, D), jnp.float32),
            pltpu.SemaphoreType.DMA,
            pltpu.SemaphoreType.DMA,
            pltpu.SemaphoreType.DMA,
            pltpu.SemaphoreType.DMA,
        ],
    )(_make_gather_body(nchunk))


def _sc_gather(qtab, ktab, dst_p, src_p, nchunk):
    return _sc_gather_kernel(nchunk)(qtab, ktab, dst_p, src_p)


def _make_gather_body(nchunk):
    gmax = _gmax(nchunk)

    def body_fn(qtab_hbm, ktab_hbm, dst_hbm, src_hbm, out_hbm,
                idall_d, idall_s, idx_d0, idx_s0, idx_d1, idx_s1,
                bq0, bk0, bq1, bk1,
                sem_g0, sem_g1, sem_w0, sem_w1):
        wid = lax.axis_index("s") * NC + lax.axis_index("c")
        start, cnt = _tile_range(wid, nchunk)
        ebase = start * CH

        pltpu.sync_copy(dst_hbm.at[pl.ds(ebase, gmax * CH)], idall_d)
        pltpu.sync_copy(src_hbm.at[pl.ds(ebase, gmax * CH)], idall_s)

        def issue(j, idx_d, idx_s, bq, bk, sem_g):
            @pl.when(j < cnt)
            def _():
                _vcopy16(idx_d, 0, idall_d, j * CH, CH)
                _vcopy16(idx_s, 0, idall_s, j * CH, CH)
                pltpu.async_copy(qtab_hbm.at[idx_d], bq, sem_g)
                pltpu.async_copy(ktab_hbm.at[idx_s], bk, sem_g)

        def process(j, idx_d, bq, bk, sem_g, sem_w):
            @pl.when((j >= 0) & (j < cnt))
            def _():
                @pl.when(j >= 2)
                def _():
                    pltpu.make_async_copy(
                        bq, out_hbm.at[pl.ds((start + j - 2) * CH, CH)], sem_w
                    ).wait()

                pltpu.make_async_copy(qtab_hbm.at[idx_d], bq, sem_g).wait()
                pltpu.make_async_copy(qtab_hbm.at[idx_d], bk, sem_g).wait()

                def add_body(i, carry2):
                    for k in range(D // 16):
                        sl = pl.ds(k * 16, 16)
                        bq[i, sl] = bq[i, sl] + bk[i, sl]
                    return carry2

                lax.fori_loop(0, CH, add_body, 0)
                pltpu.async_copy(
                    bq, out_hbm.at[pl.ds((start + j) * CH, CH)], sem_w)

        def body(t, carry):
            j0 = 2 * t
            j1 = 2 * t + 1
            issue(j0, idx_d0, idx_s0, bq0, bk0, sem_g0)
            process(j0 - 1, idx_d1, bq1, bk1, sem_g1, sem_w1)
            issue(j1, idx_d1, idx_s1, bq1, bk1, sem_g1)
            process(j0, idx_d0, bq0, bk0, sem_g0, sem_w0)
            return carry

        lax.fori_loop(0, gmax // 2 + 1, body, 0)
        pltpu.make_async_copy(
            bq0, out_hbm.at[pl.ds(ebase, CH)], sem_w0).wait()
        pltpu.make_async_copy(
            bq1, out_hbm.at[pl.ds(ebase, CH)], sem_w1).wait()

    return body_fn


_BE = 4000
_UNIT = 16000
_CHUNK_UNITS = (2, 2, 2, 2, 2)
KCH = len(_CHUNK_UNITS)


def _edge_body(pc_ref, g_ref, w1_ref, b1_ref, w2_ref, b2_ref,
               pre_ref, stats_ref):
    i = pl.program_id(0)
    pc = pc_ref[...]
    xh = lax.dot_general(pc, w1_ref[...], (((1,), (1,)), ((), ())),
                         preferred_element_type=jnp.float32)
    xh = xh + b1_ref[...]
    qh = xh[:, :D]
    kh = xh[:, D:]
    conn = jnp.maximum(g_ref[...] * qh + kh, 0.0)
    pre = lax.dot_general(conn, w2_ref[...], (((1,), (1,)), ((), ())),
                          preferred_element_type=jnp.float32)
    pre = pre + b2_ref[...] + pc
    pre_ref[...] = pre.astype(jnp.bfloat16)

    @pl.when(i == 0)
    def _():
        stats_ref[...] = jnp.zeros((8, D), jnp.float32)

    s1 = jnp.sum(pre, axis=0, keepdims=True)
    s2 = jnp.sum(pre * pre, axis=0, keepdims=True)
    stats_ref[0:1, :] = stats_ref[0:1, :] + s1
    stats_ref[1:2, :] = stats_ref[1:2, :] + s2


def _edge_stage_chunk(poly_conn, Gk, W1, b1, W2, b2, pre_full, b0, nb):
    in_specs = [
        pl.BlockSpec((_BE, D), lambda i: (i + b0, 0)),
        pl.BlockSpec((_BE, D), lambda i: (i, 0)),
        pl.BlockSpec((2 * D, D), lambda i: (0, 0)),
        pl.BlockSpec((1, 2 * D), lambda i: (0, 0)),
        pl.BlockSpec((D, D), lambda i: (0, 0)),
        pl.BlockSpec((1, D), lambda i: (0, 0)),
    ]
    inputs = (poly_conn, Gk, W1, b1, W2, b2)
    body = _edge_body
    kwargs = {}
    if b0 > 0:
        in_specs.append(pl.BlockSpec((_BE, D), lambda i: (i + b0, 0)))
        inputs = inputs + (pre_full,)
        kwargs["input_output_aliases"] = {6: 0}

        def body(pc, g, w1, b1_, w2, b2_, _pre_in, pre, st):
            _edge_body(pc, g, w1, b1_, w2, b2_, pre, st)

    return pl.pallas_call(
        body,
        grid=(nb,),
        in_specs=in_specs,
        out_specs=[
            pl.BlockSpec((_BE, D), lambda i: (i + b0, 0)),
            pl.BlockSpec((8, D), lambda i: (0, 0)),
        ],
        out_shape=[
            jax.ShapeDtypeStruct((E, D), jnp.bfloat16),
            jax.ShapeDtypeStruct((8, D), jnp.float32),
        ],
        **kwargs,
    )(*inputs)


def _bnrelu_body(pre_ref, st0, st1, st2, st3, st4, g_ref, b_ref, out_ref):
    stats = st0[...] + st1[...] + st2[...] + st3[...] + st4[...]
    inv_e = jnp.float32(1.0 / E)
    mean = stats[0:1, :] * inv_e
    ex2 = stats[1:2, :] * inv_e
    var = ex2 - mean * mean
    inv = lax.rsqrt(var + _EPS)
    scale = g_ref[...] * inv
    bias = b_ref[...] - mean * scale
    pre = pre_ref[...].astype(jnp.float32)
    out_ref[...] = jnp.maximum(pre * scale + bias, 0.0)


def _bnrelu_stage(pre, stats_list, cn_gamma, cn_beta):
    grid = (E // _BE,)
    return pl.pallas_call(
        _bnrelu_body,
        grid=grid,
        in_specs=[
            pl.BlockSpec((_BE, D), lambda i: (i, 0)),
        ] + [pl.BlockSpec((8, D), lambda i: (0, 0)) for _ in range(KCH)] + [
            pl.BlockSpec((1, D), lambda i: (0, 0)),
            pl.BlockSpec((1, D), lambda i: (0, 0)),
        ],
        out_specs=pl.BlockSpec((_BE, D), lambda i: (i, 0)),
        out_shape=jax.ShapeDtypeStruct((E, D), jnp.float32),
    )(pre, *stats_list, cn_gamma, cn_beta)


@functools.cache
def _sc_scatter_kernel():
    return functools.partial(
        pl.kernel,
        out_type=jax.ShapeDtypeStruct((NC, N, D), jnp.float32),
        mesh=_sc_mesh(),
        scratch_types=[
            pltpu.VMEM((GMAX * CH,), jnp.int32),
            pltpu.VMEM((CH,), jnp.int32),
            pltpu.VMEM((CH,), jnp.int32),
            pltpu.VMEM((CH, D), jnp.float32),
            pltpu.VMEM((CH, D), jnp.float32),
            pltpu.VMEM((CP, D), jnp.float32),
            pltpu.VMEM_SHARED((N, D), jnp.float32),
            pltpu.SemaphoreType.DMA,
            pltpu.SemaphoreType.DMA,
            pltpu.SemaphoreType.DMA,
            pltpu.SemaphoreType.DMA,
        ],
    )(_sc_scatter_body)


def _sc_scatter(conn2, dst_p):
    return _sc_scatter_kernel()(conn2, dst_p)


def _sc_scatter_body(conn2_hbm, dst_hbm, out_hbm,
                     idall, idx_v0, idx_v1, rows0, rows1, stage_v, acc_sh,
                     sem_l0, sem_l1, sem_a0, sem_a1):
    cid = lax.axis_index("c")
    sid = lax.axis_index("s")
    wid = sid * NC + cid
    nbase = pl.multiple_of(sid * NPER, 8)
    start, cnt = _tile_range(wid, NCHUNK)
    ebase = start * CH

    zero16 = jnp.zeros((16,), jnp.float32)

    def zero_body(i, carry):
        for k in range(D // 16):
            stage_v[i, pl.ds(k * 16, 16)] = zero16
        return carry

    lax.fori_loop(0, CP, zero_body, 0)

    @pl.when(sid < NS - 1)
    def _():
        for t in range(NPER // CP):
            pltpu.sync_copy(stage_v, acc_sh.at[pl.ds(nbase + t * CP, CP)])

    @pl.when(sid == NS - 1)
    def _():
        for t in range(NPER_LAST // CP):
            pltpu.sync_copy(stage_v, acc_sh.at[pl.ds(nbase + t * CP, CP)])

    pltpu.sync_copy(dst_hbm.at[pl.ds(ebase, GMAX * CH)], idall)
    plsc.subcore_barrier()

    def issue(j, rows, sem_l, idx_v, sem_a):
        @pl.when(j < cnt)
        def _():
            @pl.when(j >= 2)
            def _():
                pltpu.make_async_copy(rows, acc_sh.at[idx_v], sem_a).wait()

            pltpu.async_copy(
                conn2_hbm.at[pl.ds((start + j) * CH, CH)], rows, sem_l)

    def process(j, rows, sem_l, idx_v, sem_a):
        @pl.when((j >= 0) & (j < cnt))
        def _():
            pltpu.make_async_copy(
                conn2_hbm.at[pl.ds(ebase, CH)], rows, sem_l).wait()
            _vcopy16(idx_v, 0, idall, j * CH, CH)
            pltpu.async_copy(rows, acc_sh.at[idx_v], sem_a, add=True)

    def body(t, carry):
        j0 = 2 * t
        j1 = 2 * t + 1
        issue(j0, rows0, sem_l0, idx_v0, sem_a0)
        process(j0 - 1, rows1, sem_l1, idx_v1, sem_a1)
        issue(j1, rows1, sem_l1, idx_v1, sem_a1)
        process(j0, rows0, sem_l0, idx_v0, sem_a0)
        return carry

    lax.fori_loop(0, GMAX // 2 + 1, body, 0)
    pltpu.make_async_copy(rows0, acc_sh.at[idx_v0], sem_a0).wait()
    pltpu.make_async_copy(rows1, acc_sh.at[idx_v1], sem_a1).wait()
    plsc.subcore_barrier()

    @pl.when(sid < NS - 1)
    def _():
        for t in range(NPER // CP):
            off = nbase + t * CP
            pltpu.sync_copy(acc_sh.at[pl.ds(off, CP)], stage_v)
            pltpu.sync_copy(stage_v, out_hbm.at[cid, pl.ds(off, CP)])

    @pl.when(sid == NS - 1)
    def _():
        for t in range(NPER_LAST // CP):
            off = nbase + t * CP
            pltpu.sync_copy(acc_sh.at[pl.ds(off, CP)], stage_v)
            pltpu.sync_copy(stage_v, out_hbm.at[cid, pl.ds(off, CP)])


def _node_body(part_ref, x_ref, sdeg_ref, dc0_ref, dc1_ref,
               wf1_ref, bf1_ref, wf2_ref, bf2_ref,
               g1_ref, b1n_ref, g2_ref, b2n_ref, out_ref):
    nh = part_ref[0] + part_ref[1]
    sdeg = sdeg_ref[...]
    nh = nh * (dc0_ref[...] + sdeg * dc1_ref[...])
    h_res = nh + x_ref[...]

    m1 = jnp.mean(h_res, axis=0, keepdims=True)
    hc = h_res - m1
    v1 = jnp.mean(hc * hc, axis=0, keepdims=True)
    nh1 = g1_ref[...] * hc * lax.rsqrt(v1 + _EPS) + b1n_ref[...]

    t = lax.dot_general(nh1, wf1_ref[...], (((1,), (1,)), ((), ())),
                        preferred_element_type=jnp.float32)
    t = jnp.maximum(t + bf1_ref[...], 0.0)
    t2 = lax.dot_general(t, wf2_ref[...], (((1,), (1,)), ((), ())),
                         preferred_element_type=jnp.float32)
    h2 = t2 + bf2_ref[...] + h_res

    m2 = jnp.mean(h2, axis=0, keepdims=True)
    hc2 = h2 - m2
    v2 = jnp.mean(hc2 * hc2, axis=0, keepdims=True)
    out_ref[...] = g2_ref[...] * hc2 * lax.rsqrt(v2 + _EPS) + b2n_ref[...]


def _node_stage(parts, x, sdeg, dc0, dc1, Wf1, bf1, Wf2, bf2,
                n1_gamma, n1_beta, n2_gamma, n2_beta):
    return pl.pallas_call(
        _node_body,
        in_specs=[
            pl.BlockSpec((NC, N, D), lambda: (0, 0, 0)),
            pl.BlockSpec((N, D), lambda: (0, 0)),
            pl.BlockSpec((N, 1), lambda: (0, 0)),
            pl.BlockSpec((1, D), lambda: (0, 0)),
            pl.BlockSpec((1, D), lambda: (0, 0)),
            pl.BlockSpec((2 * D, D), lambda: (0, 0)),
            pl.BlockSpec((1, 2 * D), lambda: (0, 0)),
            pl.BlockSpec((D, 2 * D), lambda: (0, 0)),
            pl.BlockSpec((1, D), lambda: (0, 0)),
            pl.BlockSpec((1, D), lambda: (0, 0)),
            pl.BlockSpec((1, D), lambda: (0, 0)),
            pl.BlockSpec((1, D), lambda: (0, 0)),
            pl.BlockSpec((1, D), lambda: (0, 0)),
        ],
        out_specs=pl.BlockSpec((N, D), lambda: (0, 0)),
        out_shape=jax.ShapeDtypeStruct((N, D), jnp.float32),
    )(parts, x, sdeg, dc0, dc1, Wf1, bf1, Wf2, bf2,
      n1_gamma, n1_beta, n2_gamma, n2_beta)


def kernel(x, poly_conn, sqrt_deg, W1, b1, W2, b2, cn_gamma, cn_beta,
           deg_coef, Wf1, bf1, Wf2, bf2, n1_gamma, n1_beta, n2_gamma,
           n2_beta, poly_index):
    dst = poly_index[0].astype(jnp.int32)
    src = poly_index[1].astype(jnp.int32)
    pad = jnp.zeros((CH,), jnp.int32)
    dst_p = jnp.concatenate([dst, pad])
    src_p = jnp.concatenate([src, pad])

    b1r = b1.reshape(1, 2 * D)
    b2r = b2.reshape(1, D)
    bf1r = bf1.reshape(1, 2 * D)
    bf2r = bf2.reshape(1, D)
    cng = cn_gamma.reshape(1, D)
    cnb = cn_beta.reshape(1, D)
    dc0 = deg_coef[:, :, 0]
    dc1 = deg_coef[:, :, 1]

    qtab, ktab = _make_tables(poly_conn, W1, b1r)
    Gs = []
    off = 0
    for u in _CHUNK_UNITS:
        n_e = u * _UNIT
        Gs.append(_sc_gather(qtab, ktab, dst_p[off:off + n_e + CH],
                             src_p[off:off + n_e + CH], n_e // CH))
        off += n_e
    pre = None
    stats_list = []
    off = 0
    for k, u in enumerate(_CHUNK_UNITS):
        n_e = u * _UNIT
        pre, st = _edge_stage_chunk(poly_conn, Gs[k], W1, b1r, W2, b2r, pre,
                                    off // _BE, n_e // _BE)
        stats_list.append(st)
        off += n_e
    conn2 = _bnrelu_stage(pre, stats_list, cng, cnb)
    parts = _sc_scatter(conn2, dst_p)
    nh = _node_stage(parts, x, sqrt_deg, dc0, dc1, Wf1, bf1r, Wf2, bf2r,
                     n1_gamma.reshape(1, D), n1_beta.reshape(1, D),
                     n2_gamma.reshape(1, D), n2_beta.reshape(1, D))
    return nh, conn2
